# Initial kernel scaffold; baseline (speedup 1.0000x reference)
import jax
import jax.numpy as jnp
from jax.experimental import pallas as pl


def kernel(feat, edge_index, W0, b0, W1, b1, W2, b2):
    raise NotImplementedError("write your pallas kernel here")


# speedup vs baseline: 3.5652x; 3.5652x over previous
import functools

import jax
import jax.numpy as jnp
from jax import lax
from jax.experimental import pallas as pl
from jax.experimental.pallas import tpu as pltpu
from jax.experimental.pallas import tpu_sc as plsc

N = 10000
E = 320000
D = 128
NCLS = 40

NP = 10240
NCORES = 2
NSUB = 16
NW = NCORES * NSUB
CH = 128

STEPS = 79
EPT = STEPS * CH
EP = NW * EPT

DR = 2 * NP
NSTRIPE = NP // NSUB

RB = 512
GRID = NP // RB
DBLK = 1024
GROUPS = CH // 16


def _sc_mesh():
    return plsc.VectorSubcoreMesh(core_axis_name="c", subcore_axis_name="s")


def _degree_sc(src_p, dst_p, zcnt):
    @functools.partial(
        pl.kernel,
        out_type=jax.ShapeDtypeStruct((NCORES, NSUB, DR), jnp.float32),
        mesh=_sc_mesh(),
        scratch_types=[
            pltpu.VMEM((CH,), jnp.int32),
            pltpu.VMEM((CH,), jnp.int32),
            pltpu.VMEM((DR,), jnp.float32),
        ],
        compiler_params=pltpu.CompilerParams(needs_layout_passes=False),
    )
    def k(src_hbm, dst_hbm, zcnt_hbm, out_hbm, sidx_v, didx_v, cnt_v):
        c = lax.axis_index("c")
        s = lax.axis_index("s")
        wid = c * NSUB + s
        pltpu.sync_copy(zcnt_hbm, cnt_v)
        ones = jnp.full((16,), 1.0, jnp.float32)
        npoff = jnp.full((16,), NP, jnp.int32)

        def body(step, carry):
            base = wid * EPT + step * CH
            pltpu.sync_copy(src_hbm.at[pl.ds(base, CH)], sidx_v)
            pltpu.sync_copy(dst_hbm.at[pl.ds(base, CH)], didx_v)
            for g in range(GROUPS):
                iv = sidx_v[pl.ds(g * 16, 16)]
                plsc.addupdate_scatter(cnt_v, [iv], ones)
                jv = didx_v[pl.ds(g * 16, 16)] + npoff
                plsc.addupdate_scatter(cnt_v, [jv], ones)
            return carry

        lax.fori_loop(0, STEPS, body, 0)
        pltpu.sync_copy(cnt_v, out_hbm.at[c, s])

    return k(src_p, dst_p, zcnt)


def _aggregate_sc(table, src_p, dst_p, zagg):
    @functools.partial(
        pl.kernel,
        out_type=jax.ShapeDtypeStruct((NCORES, NP, D), jnp.float32),
        mesh=_sc_mesh(),
        scratch_types=[
            pltpu.VMEM((CH,), jnp.int32),
            pltpu.VMEM((CH,), jnp.int32),
            pltpu.VMEM((CH, D), jnp.float32),
            pltpu.VMEM_SHARED((NP, D), jnp.float32),
            pltpu.SemaphoreType.DMA,
        ],
    )
    def k(table_hbm, src_hbm, dst_hbm, zagg_hbm, out_hbm,
          sidx_v, didx_v, rows_v, agg_sh, sem):
        c = lax.axis_index("c")
        s = lax.axis_index("s")
        wid = c * NSUB + s
        pltpu.sync_copy(zagg_hbm, agg_sh.at[pl.ds(s * NSTRIPE, NSTRIPE)])
        plsc.subcore_barrier()

        def body(step, carry):
            base = wid * EPT + step * CH
            pltpu.sync_copy(src_hbm.at[pl.ds(base, CH)], sidx_v)
            pltpu.async_copy(table_hbm.at[sidx_v], rows_v, sem).wait()
            pltpu.sync_copy(dst_hbm.at[pl.ds(base, CH)], didx_v)
            pltpu.sync_copy(rows_v, agg_sh.at[didx_v], add=True)
            return carry

        lax.fori_loop(0, STEPS, body, 0)
        plsc.subcore_barrier()
        pltpu.sync_copy(
            agg_sh.at[pl.ds(s * NSTRIPE, NSTRIPE)],
            out_hbm.at[c, pl.ds(s * NSTRIPE, NSTRIPE)],
        )

    return k(table, src_p, dst_p, zagg)


def _norms_tc(degparts):
    def body(deg_ref, out_ref):
        d = jnp.sum(deg_ref[...], axis=0, keepdims=True)
        out_ref[...] = lax.rsqrt(jnp.maximum(d, 1.0))

    return pl.pallas_call(
        body,
        grid=(DR // DBLK,),
        in_specs=[pl.BlockSpec((NW, DBLK), lambda i: (0, i))],
        out_specs=pl.BlockSpec((1, DBLK), lambda i: (0, i)),
        out_shape=jax.ShapeDtypeStruct((1, DR), jnp.float32),
    )(degparts)


def _prep_tc(featp, ns_col):
    def body(feat_ref, ns_ref, out_ref):
        out_ref[...] = feat_ref[...] * ns_ref[...]

    return pl.pallas_call(
        body,
        grid=(GRID,),
        in_specs=[
            pl.BlockSpec((RB, D), lambda i: (i, 0)),
            pl.BlockSpec((RB, 1), lambda i: (i, 0)),
        ],
        out_specs=pl.BlockSpec((RB, D), lambda i: (i, 0)),
        out_shape=jax.ShapeDtypeStruct((NP, D), jnp.float32),
    )(featp, ns_col)


def _layer_tc(aggparts, nd_col, ns_col, W, b, relu, want_h, want_table):
    def body(agg_ref, nd_ref, ns_ref, w_ref, b_ref, *out_refs):
        agg = agg_ref[0] + agg_ref[1]
        z = jnp.dot(agg * nd_ref[...], w_ref[...],
                    preferred_element_type=jnp.float32)
        z = z + b_ref[...]
        if relu:
            z = jnp.maximum(z, 0.0)
        o = 0
        if want_h:
            out_refs[o][...] = z
            o += 1
        if want_table:
            out_refs[o][...] = z * ns_ref[...]

    out_specs = []
    out_shape = []
    for flag in (want_h, want_table):
        if flag:
            out_specs.append(pl.BlockSpec((RB, D), lambda i: (i, 0)))
            out_shape.append(jax.ShapeDtypeStruct((NP, D), jnp.float32))

    return pl.pallas_call(
        body,
        grid=(GRID,),
        in_specs=[
            pl.BlockSpec((NCORES, RB, D), lambda i: (0, i, 0)),
            pl.BlockSpec((RB, 1), lambda i: (i, 0)),
            pl.BlockSpec((RB, 1), lambda i: (i, 0)),
            pl.BlockSpec((D, D), lambda i: (0, 0)),
            pl.BlockSpec((1, D), lambda i: (0, 0)),
        ],
        out_specs=out_specs,
        out_shape=out_shape,
    )(aggparts, nd_col, ns_col, W, b)


def kernel(feat, edge_index, W0, b0, W1, b1, W2, b2):
    src = edge_index[0]
    dst = edge_index[1]

    pad_e = jnp.full((EP - E,), N, dtype=jnp.int32)
    src_p = jnp.concatenate([src, pad_e])
    dst_p = jnp.concatenate([dst, pad_e])
    featp = jnp.pad(feat, ((0, NP - N), (0, 0)))
    zcnt = jnp.zeros((DR,), jnp.float32)
    zagg = jnp.zeros((NSTRIPE, D), jnp.float32)
    W2p = jnp.pad(W2, ((0, 0), (0, D - NCLS)))
    b0r = b0.reshape(1, D)
    b1r = b1.reshape(1, D)
    b2r = jnp.pad(b2, (0, D - NCLS)).reshape(1, D)

    degparts = _degree_sc(src_p, dst_p, zcnt).reshape(NW, DR)
    norms = _norms_tc(degparts).reshape(DR)
    ns_col = norms[0:NP].reshape(NP, 1)
    nd_col = norms[NP:2 * NP].reshape(NP, 1)

    table0 = _prep_tc(featp, ns_col)
    agg0 = _aggregate_sc(table0, src_p, dst_p, zagg)
    (table1,) = _layer_tc(agg0, nd_col, ns_col, W0, b0r,
                          relu=True, want_h=False, want_table=True)

    agg1 = _aggregate_sc(table1, src_p, dst_p, zagg)
    h1, table2 = _layer_tc(agg1, nd_col, ns_col, W1, b1r,
                           relu=True, want_h=True, want_table=True)

    agg2 = _aggregate_sc(table2, src_p, dst_p, zagg)
    (out,) = _layer_tc(agg2, nd_col, ns_col, W2p, b2r,
                       relu=False, want_h=True, want_table=False)

    return (out[:N, :NCLS], h1[:N])
